# baseline (speedup 1.0000x reference)
import functools

import jax
import jax.numpy as jnp
from jax import lax
from jax.experimental import pallas as pl
from jax.experimental.pallas import tpu as pltpu
from jax.experimental.pallas import tpu_sc as plsc

_B = 1024
_NBR = 16
_DIM = 128
_NW = 32
_BPW = _B // _NW
_ROWS = _BPW * _NBR
_NCHUNK = 8
_CROWS = _ROWS // _NCHUNK
_SPC = _CROWS // _NBR
_LANES = 16


def _sc_body(v_hbm, adjT_hbm, ent_hbm, out_hbm, vidx, blocks, flat, rows,
             outbuf, vsm, sema, semb, sem0, sem1, sem2, sem3):
    wid = lax.axis_index("s") * 2 + lax.axis_index("c")
    base = wid * _BPW

    pltpu.sync_copy(v_hbm.at[pl.ds(base, _BPW)], vidx)
    for t in range(_BPW // _LANES):
        v16 = vidx[pl.ds(t * _LANES, _LANES)]
        for u in range(_LANES):
            vsm[t * _LANES + u] = v16[u]

    lane = lax.iota(jnp.int32, _LANES)
    wave_sems = [sema, semb]
    ent_sems = [sem0, sem1, sem2, sem3]

    def fire_wave(w):
        def fbody(u, carry):
            colbase = (vsm[w * _SPC + u] >> 7) * 128
            pltpu.async_copy(
                adjT_hbm.at[:, pl.ds(colbase, 128)],
                blocks.at[(w % 2) * _SPC + u], wave_sems[w % 2],
            )
            return carry
        lax.fori_loop(0, _SPC, fbody, 0)

    def drain_wave(w):
        def dbody(u, carry):
            pltpu.make_async_copy(
                adjT_hbm.at[:, pl.ds(0, 128)], blocks.at[0], wave_sems[w % 2]
            ).wait()
            return carry
        lax.fori_loop(0, _SPC, dbody, 0)

    def extract_wave(w):
        def ebody(u, carry):
            vk = vsm[w * _SPC + u]
            c0 = vk & 127
            cbase = c0 & 112
            l0 = jnp.full((_LANES,), c0 & 15, jnp.int32)
            res = jnp.zeros((_LANES,), jnp.int32)
            for j in range(_NBR):
                bv = blocks[(w % 2) * _SPC + u, j, pl.ds(cbase, _LANES)]
                t = bv.at[l0].get(mode="promise_in_bounds")
                res = jnp.where(lane == j, t, res)
            flat[w, pl.ds(u * _NBR, _NBR)] = res
            return carry
        lax.fori_loop(0, _SPC, ebody, 0)

    def fire_ent(c):
        return pltpu.async_copy(
            ent_hbm.at[flat.at[c]],
            rows.at[pl.ds((c % 2) * _CROWS, _CROWS)],
            ent_sems[c % len(ent_sems)],
        )

    def reduce_chunk(c):
        def body(i, carry):
            r0 = (c % 2) * _CROWS + i * _NBR
            for d in range(_DIM // _LANES):
                sl = pl.ds(d * _LANES, _LANES)
                acc = rows[r0, sl]
                for j in range(1, _NBR):
                    acc = acc + rows[r0 + j, sl]
                outbuf[c * _SPC + i, sl] = acc * (1.0 / _NBR)
            return carry
        lax.fori_loop(0, _SPC, body, 0)

    fire_wave(0)
    ent_copies = []
    for w in range(_NCHUNK):
        drain_wave(w)
        extract_wave(w)
        ent_copies.append(fire_ent(w))
        if w + 1 < _NCHUNK:
            fire_wave(w + 1)
        if w >= 1:
            ent_copies[w - 1].wait()
            reduce_chunk(w - 1)
    ent_copies[_NCHUNK - 1].wait()
    reduce_chunk(_NCHUNK - 1)

    pltpu.sync_copy(outbuf, out_hbm.at[pl.ds(base, _BPW)])


@jax.jit
def kernel(v, adj_ent, ent):
    v = v.astype(jnp.int32)
    adjT = adj_ent.astype(jnp.int32).T
    ent = ent.astype(jnp.float32)

    mesh = plsc.VectorSubcoreMesh(core_axis_name="c", subcore_axis_name="s")
    run = functools.partial(
        pl.kernel,
        mesh=mesh,
        out_type=jax.ShapeDtypeStruct((_B, _DIM), jnp.float32),
        scratch_types=[
            pltpu.VMEM((_BPW,), jnp.int32),
            pltpu.VMEM((2 * _SPC, _NBR, 128), jnp.int32),
            pltpu.VMEM((_NCHUNK, _CROWS), jnp.int32),
            pltpu.VMEM((2 * _CROWS, _DIM), jnp.float32),
            pltpu.VMEM((_BPW, _DIM), jnp.float32),
            pltpu.SMEM((_BPW,), jnp.int32),
            pltpu.SemaphoreType.DMA,
            pltpu.SemaphoreType.DMA,
            pltpu.SemaphoreType.DMA,
            pltpu.SemaphoreType.DMA,
            pltpu.SemaphoreType.DMA,
            pltpu.SemaphoreType.DMA,
        ],
    )(_sc_body)
    return run(v, adjT, ent)


# speedup vs baseline: 1.0732x; 1.0732x over previous
import functools

import jax
import jax.numpy as jnp
from jax import lax
from jax.experimental import pallas as pl
from jax.experimental.pallas import tpu as pltpu
from jax.experimental.pallas import tpu_sc as plsc

_B = 1024
_NBR = 16
_DIM = 128
_NW = 32
_BPW = _B // _NW
_ROWS = _BPW * _NBR
_NCHUNK = 4
_CROWS = _ROWS // _NCHUNK
_SPC = _CROWS // _NBR
_WPW = 2
_SPW = _BPW // _WPW
_LANES = 16


def _sc_body(v_hbm, adjT_hbm, ent_hbm, out_hbm, vidx, blocks, flat, rows,
             outbuf, vsm, sema, semb, sem0, sem1, sem2, sem3):
    wid = lax.axis_index("s") * 2 + lax.axis_index("c")
    base = wid * _BPW

    pltpu.sync_copy(v_hbm.at[pl.ds(base, _BPW)], vidx)
    for t in range(_BPW // _LANES):
        v16 = vidx[pl.ds(t * _LANES, _LANES)]
        for u in range(_LANES):
            vsm[t * _LANES + u] = v16[u]

    lane = lax.iota(jnp.int32, _LANES)
    wave_sems = [sema, semb]
    ent_sems = [sem0, sem1, sem2, sem3]

    def fire_wave(w):
        def fbody(u, carry):
            colbase = (vsm[w * _SPW + u] >> 7) * 128
            pltpu.async_copy(
                adjT_hbm.at[:, pl.ds(colbase, 128)],
                blocks.at[(w % 2) * _SPW + u], wave_sems[w % 2],
            )
            return carry
        lax.fori_loop(0, _SPW, fbody, 0)

    def drain_wave(w):
        def dbody(u, carry):
            pltpu.make_async_copy(
                adjT_hbm.at[:, pl.ds(0, 128)], blocks.at[0], wave_sems[w % 2]
            ).wait()
            return carry
        lax.fori_loop(0, _SPW, dbody, 0)

    def extract_wave(w):
        def ebody(u, carry):
            k = w * _SPW + u
            vk = vsm[k]
            c0 = vk & 127
            cbase = c0 & 112
            l0 = jnp.full((_LANES,), c0 & 15, jnp.int32)
            res = jnp.zeros((_LANES,), jnp.int32)
            for j in range(_NBR):
                bv = blocks[(w % 2) * _SPW + u, j, pl.ds(cbase, _LANES)]
                t = bv.at[l0].get(mode="promise_in_bounds")
                res = jnp.where(lane == j, t, res)
            flat[k // _SPC, pl.ds((k % _SPC) * _NBR, _NBR)] = res
            return carry
        lax.fori_loop(0, _SPW, ebody, 0)

    def fire_ent(c):
        return pltpu.async_copy(
            ent_hbm.at[flat.at[c]],
            rows.at[pl.ds((c % 2) * _CROWS, _CROWS)],
            ent_sems[c % len(ent_sems)],
        )

    def reduce_chunk(c):
        def body(i, carry):
            r0 = (c % 2) * _CROWS + i * _NBR
            for d in range(_DIM // _LANES):
                sl = pl.ds(d * _LANES, _LANES)
                acc = rows[r0, sl]
                for j in range(1, _NBR):
                    acc = acc + rows[r0 + j, sl]
                outbuf[c * _SPC + i, sl] = acc * (1.0 / _NBR)
            return carry
        lax.fori_loop(0, _SPC, body, 0)

    fire_wave(0)
    drain_wave(0)
    extract_wave(0)
    ent_copies = [fire_ent(0), fire_ent(1)]
    fire_wave(1)
    ent_copies[0].wait()
    reduce_chunk(0)
    ent_copies[1].wait()
    reduce_chunk(1)
    drain_wave(1)
    extract_wave(1)
    ent_copies += [fire_ent(2), fire_ent(3)]
    ent_copies[2].wait()
    reduce_chunk(2)
    ent_copies[3].wait()
    reduce_chunk(3)

    pltpu.sync_copy(outbuf, out_hbm.at[pl.ds(base, _BPW)])


@jax.jit
def kernel(v, adj_ent, ent):
    v = v.astype(jnp.int32)
    adjT = adj_ent.astype(jnp.int32).T
    ent = ent.astype(jnp.float32)

    mesh = plsc.VectorSubcoreMesh(core_axis_name="c", subcore_axis_name="s")
    run = functools.partial(
        pl.kernel,
        mesh=mesh,
        out_type=jax.ShapeDtypeStruct((_B, _DIM), jnp.float32),
        scratch_types=[
            pltpu.VMEM((_BPW,), jnp.int32),
            pltpu.VMEM((2 * _SPW, _NBR, 128), jnp.int32),
            pltpu.VMEM((_NCHUNK, _CROWS), jnp.int32),
            pltpu.VMEM((2 * _CROWS, _DIM), jnp.float32),
            pltpu.VMEM((_BPW, _DIM), jnp.float32),
            pltpu.SMEM((_BPW,), jnp.int32),
            pltpu.SemaphoreType.DMA,
            pltpu.SemaphoreType.DMA,
            pltpu.SemaphoreType.DMA,
            pltpu.SemaphoreType.DMA,
            pltpu.SemaphoreType.DMA,
            pltpu.SemaphoreType.DMA,
        ],
    )(_sc_body)
    return run(v, adjT, ent)


# speedup vs baseline: 1.1279x; 1.0511x over previous
import functools

import jax
import jax.numpy as jnp
from jax import lax
from jax.experimental import pallas as pl
from jax.experimental.pallas import tpu as pltpu
from jax.experimental.pallas import tpu_sc as plsc

_B = 1024
_NBR = 16
_DIM = 128
_NW = 32
_BPW = _B // _NW
_ROWS = _BPW * _NBR
_NCHUNK = 4
_CROWS = _ROWS // _NCHUNK
_SPC = _CROWS // _NBR
_WPW = 2
_SPW = _BPW // _WPW
_LANES = 16


def _sc_body(v_hbm, adjT_hbm, ent_hbm, out_hbm, vidx, blocks, flat, rows,
             outbuf, vsm, sema, semb, sem0, sem1, sem2, sem3):
    wid = lax.axis_index("s") * 2 + lax.axis_index("c")
    base = wid * _BPW

    pltpu.sync_copy(v_hbm.at[pl.ds(base, _BPW)], vidx)
    for t in range(_BPW // _LANES):
        v16 = vidx[pl.ds(t * _LANES, _LANES)]
        for u in range(_LANES):
            vsm[t * _LANES + u] = v16[u]

    lane = lax.iota(jnp.int32, _LANES)
    wave_sems = [sema, semb]
    ent_sems = [sem0, sem1, sem2, sem3]

    def fire_wave(w):
        def fbody(u, carry):
            colbase = (vsm[w * _SPW + u] >> 7) * 128
            pltpu.async_copy(
                adjT_hbm.at[:, pl.ds(colbase, 128)],
                blocks.at[(w % 2) * _SPW + u], wave_sems[w % 2],
            )
            return carry
        lax.fori_loop(0, _SPW, fbody, 0)

    def drain_wave(w):
        def dbody(u, carry):
            pltpu.make_async_copy(
                adjT_hbm.at[:, pl.ds(0, 128)], blocks.at[0], wave_sems[w % 2]
            ).wait()
            return carry
        lax.fori_loop(0, _SPW, dbody, 0)

    def extract_wave(w):
        def ebody(u, carry):
            k = w * _SPW + u
            vk = vsm[k]
            c0 = vk & 127
            cbase = c0 & 112
            l0 = jnp.full((_LANES,), c0 & 15, jnp.int32)
            res = jnp.zeros((_LANES,), jnp.int32)
            for j in range(_NBR):
                bv = blocks[(w % 2) * _SPW + u, j, pl.ds(cbase, _LANES)]
                t = bv.at[l0].get(mode="promise_in_bounds")
                res = jnp.where(lane == j, t, res)
            flat[k // _SPC, pl.ds((k % _SPC) * _NBR, _NBR)] = res
            return carry
        lax.fori_loop(0, _SPW, ebody, 0)

    def fire_ent(c):
        return pltpu.async_copy(
            ent_hbm.at[flat.at[c]],
            rows.at[pl.ds((c % 2) * _CROWS, _CROWS)],
            ent_sems[c % len(ent_sems)],
        )

    def reduce_chunk(c):
        def body(i, carry):
            r0 = (c % 2) * _CROWS + i * _NBR
            for d in range(_DIM // _LANES):
                sl = pl.ds(d * _LANES, _LANES)
                acc = rows[r0, sl]
                for j in range(1, _NBR):
                    acc = acc + rows[r0 + j, sl]
                outbuf[c * _SPC + i, sl] = acc * (1.0 / _NBR)
            return carry
        lax.fori_loop(0, _SPC, body, 0)

    fire_wave(0)
    fire_wave(1)
    drain_wave(0)
    extract_wave(0)
    ent_copies = [fire_ent(0), fire_ent(1)]
    drain_wave(1)
    extract_wave(1)

    ent_copies[0].wait()
    reduce_chunk(0)
    ent_copies.append(fire_ent(2))
    ent_copies[1].wait()
    reduce_chunk(1)
    ent_copies.append(fire_ent(3))
    ent_copies[2].wait()
    reduce_chunk(2)
    ent_copies[3].wait()
    reduce_chunk(3)

    pltpu.sync_copy(outbuf, out_hbm.at[pl.ds(base, _BPW)])


@jax.jit
def kernel(v, adj_ent, ent):
    v = v.astype(jnp.int32)
    adjT = adj_ent.astype(jnp.int32).T
    ent = ent.astype(jnp.float32)

    mesh = plsc.VectorSubcoreMesh(core_axis_name="c", subcore_axis_name="s")
    run = functools.partial(
        pl.kernel,
        mesh=mesh,
        out_type=jax.ShapeDtypeStruct((_B, _DIM), jnp.float32),
        scratch_types=[
            pltpu.VMEM((_BPW,), jnp.int32),
            pltpu.VMEM((2 * _SPW, _NBR, 128), jnp.int32),
            pltpu.VMEM((_NCHUNK, _CROWS), jnp.int32),
            pltpu.VMEM((2 * _CROWS, _DIM), jnp.float32),
            pltpu.VMEM((_BPW, _DIM), jnp.float32),
            pltpu.SMEM((_BPW,), jnp.int32),
            pltpu.SemaphoreType.DMA,
            pltpu.SemaphoreType.DMA,
            pltpu.SemaphoreType.DMA,
            pltpu.SemaphoreType.DMA,
            pltpu.SemaphoreType.DMA,
            pltpu.SemaphoreType.DMA,
        ],
    )(_sc_body)
    return run(v, adjT, ent)
